# Initial kernel scaffold; baseline (speedup 1.0000x reference)
import jax
import jax.numpy as jnp
from jax.experimental import pallas as pl


def kernel(edm_true, edm_pred, dy_pred, dx_pred, cat_pred, cat_true, labels):
    raise NotImplementedError("write your pallas kernel here")


# speedup vs baseline: 93.0296x; 93.0296x over previous
import functools

import jax
import jax.numpy as jnp
from jax import lax
from jax.experimental import pallas as pl
from jax.experimental.pallas import tpu as pltpu

B, H, W = 4, 384, 384
NLAB = 32
BH_A = 96
BH_B = 96
SUMW = 40


def _sums_kernel(labels_ref, dy_ref, dx_ref, out_ref):
    t = pl.program_id(1)

    @pl.when(t == 0)
    def _init():
        for q in range(3):
            for l in range(SUMW):
                out_ref[0, q, l] = 0.0

    labels = labels_ref[0]
    dy = dy_ref[0]
    dx = dx_ref[0]
    for l in range(1, NLAB + 1):
        m = labels == l
        cnt = jnp.sum(m.astype(jnp.float32))
        sdy = jnp.sum(jnp.where(m, dy, 0.0))
        sdx = jnp.sum(jnp.where(m, dx, 0.0))
        out_ref[0, 0, l] += cnt
        out_ref[0, 1, l] += sdy
        out_ref[0, 2, l] += sdx


def _loss_kernel(sums_ref, edm_t_ref, edm_p_ref, dy_ref, dx_ref, catp_ref,
                 cat_t_ref, labels_ref, out_ref):
    labels = labels_ref[0]
    dy = dy_ref[0]
    dx = dx_ref[0]

    dym = jnp.zeros_like(dy)
    dxm = jnp.zeros_like(dx)
    for l in range(1, NLAB + 1):
        cnt = sums_ref[0, 0, l]
        inv = 1.0 / jnp.maximum(cnt, 1.0)
        mdy = sums_ref[0, 1, l] * inv
        mdx = sums_ref[0, 2, l] * inv
        m = labels == l
        dym = jnp.where(m, mdy, dym)
        dxm = jnp.where(m, mdx, dxm)

    edm_l = jnp.square(edm_t_ref[0] - edm_p_ref[0])
    dm_l = jnp.square(dym - dy) + jnp.square(dxm - dx)

    ct = cat_t_ref[0]
    c0 = catp_ref[0, 0]
    c1 = catp_ref[0, 1]
    c2 = catp_ref[0, 2]
    c3 = catp_ref[0, 3]
    s = c0 + c1 + c2 + c3
    pt = jnp.where(ct == 1, c1, c0)
    pt = jnp.where(ct == 2, c2, pt)
    pt = jnp.where(ct == 3, c3, pt)
    p = jnp.clip(pt / s, 1e-7, 1.0 - 1e-7)
    w = jnp.where(ct >= 2, 5.0, 1.0)
    cat_l = -jnp.log(p) * w

    out_ref[0] = edm_l * (1.0 / 3.0) + dm_l * (1.0 / 6.0) + cat_l * (1.0 / 3.0)


@jax.jit
def kernel(edm_true, edm_pred, dy_pred, dx_pred, cat_pred, cat_true, labels):
    edm_t = edm_true.reshape(B, H, W)
    edm_p = edm_pred.reshape(B, H, W)
    dy = dy_pred.reshape(B, H, W)
    dx = dx_pred.reshape(B, H, W)
    ct = cat_true.reshape(B, H, W)
    lab = labels.reshape(B, H, W)
    catp = jnp.transpose(cat_pred, (0, 3, 1, 2))

    nt_a = H // BH_A
    sums = pl.pallas_call(
        _sums_kernel,
        grid=(B, nt_a),
        in_specs=[
            pl.BlockSpec((1, BH_A, W), lambda b, t: (b, t, 0)),
            pl.BlockSpec((1, BH_A, W), lambda b, t: (b, t, 0)),
            pl.BlockSpec((1, BH_A, W), lambda b, t: (b, t, 0)),
        ],
        out_specs=pl.BlockSpec((1, 3, SUMW), lambda b, t: (b, 0, 0),
                               memory_space=pltpu.SMEM),
        out_shape=jax.ShapeDtypeStruct((B, 3, SUMW), jnp.float32),
    )(lab, dy, dx)

    nt_b = H // BH_B
    loss = pl.pallas_call(
        _loss_kernel,
        grid=(B, nt_b),
        in_specs=[
            pl.BlockSpec((1, 3, SUMW), lambda b, t: (b, 0, 0),
                         memory_space=pltpu.SMEM),
            pl.BlockSpec((1, BH_B, W), lambda b, t: (b, t, 0)),
            pl.BlockSpec((1, BH_B, W), lambda b, t: (b, t, 0)),
            pl.BlockSpec((1, BH_B, W), lambda b, t: (b, t, 0)),
            pl.BlockSpec((1, BH_B, W), lambda b, t: (b, t, 0)),
            pl.BlockSpec((1, 4, BH_B, W), lambda b, t: (b, 0, t, 0)),
            pl.BlockSpec((1, BH_B, W), lambda b, t: (b, t, 0)),
            pl.BlockSpec((1, BH_B, W), lambda b, t: (b, t, 0)),
        ],
        out_specs=pl.BlockSpec((1, BH_B, W), lambda b, t: (b, t, 0)),
        out_shape=jax.ShapeDtypeStruct((B, H, W), jnp.float32),
    )(sums, edm_t, edm_p, dy, dx, catp, ct, lab)
    return loss
